# baseline (speedup 1.0000x reference)
import functools

import jax
import jax.numpy as jnp
from jax.experimental import pallas as pl
from jax.experimental.pallas import tpu as pltpu
from jax.experimental.pallas import tpu_sc as plsc

V = 32000
D = 1024
L = 4
E = 8
DFF = 512
NSEQ = 2048
H = D // 2
THRESH = 0.99

TB = 256
NB = NSEQ // TB
VB = 1280
NVB = V // VB
GW = 128
SUB = 8
CW = D // SUB


def _bdot(x, w):
    return jnp.dot(x.astype(jnp.bfloat16), w,
                   preferred_element_type=jnp.float32)


def _sc_gather(tok_emb, idx):
    mesh = plsc.VectorSubcoreMesh(core_axis_name="core", subcore_axis_name="subcore")
    te = tok_emb.reshape(V * SUB, CW)
    idx8 = (idx.reshape(-1, 1) * SUB
            + jnp.arange(SUB, dtype=jnp.int32)).reshape(1, NSEQ * SUB)

    @functools.partial(
        pl.kernel,
        out_type=jax.ShapeDtypeStruct((NSEQ * SUB, CW), tok_emb.dtype),
        mesh=mesh,
    )
    def kern(x_hbm, i_hbm, o_hbm):
        def body(i_vmem, o_vmem):
            pltpu.sync_copy(x_hbm.at[i_vmem.at[0]], o_vmem)

        pltpu.emit_pipeline(
            body,
            grid=(NSEQ * SUB // GW,),
            in_specs=[pl.BlockSpec((1, GW), lambda i: (0, i))],
            out_specs=[pl.BlockSpec((GW, CW), lambda i: (i, 0))],
            core_axis_name=("core", "subcore"),
            dimension_semantics=(pltpu.PARALLEL,),
        )(i_hbm, o_hbm)

    return kern(te, idx8).reshape(NSEQ, D)


def _layer_body(first, emb_ref, pos_ref, cum_ref, run_ref, acc_ref,
                lng_ref, lnb_ref, rw_ref, rb_ref, w1_ref, b1_ref,
                w2_ref, b2_ref, hw1_ref, hb1_ref, hw2_ref, hb2_ref,
                h_out, cum_out, run_out, acc_out, w_out):
    if first:
        h = emb_ref[...] + pos_ref[...]
        cum = jnp.zeros((TB, 1), jnp.float32)
        run = jnp.ones((TB, 1), jnp.float32)
        acc = jnp.zeros((TB, D), jnp.float32)
    else:
        h = emb_ref[...]
        cum = cum_ref[...]
        run = run_ref[...]
        acc = acc_ref[...]

    m = jnp.mean(h, axis=-1, keepdims=True)
    c = h - m
    v = jnp.mean(c * c, axis=-1, keepdims=True)
    xn = c / jnp.sqrt(v + 1e-5) * lng_ref[...] + lnb_ref[...]

    glog = _bdot(xn, rw_ref[...]) + rb_ref[...]
    gmax = jnp.max(glog, axis=-1, keepdims=True)
    ge = jnp.exp(glog - gmax)
    gates = ge / jnp.sum(ge, axis=-1, keepdims=True)

    lane = jax.lax.broadcasted_iota(jnp.int32, (TB, E), 1)
    v1 = jnp.max(gates, axis=-1, keepdims=True)
    i1 = jnp.min(jnp.where(gates == v1, lane, E), axis=-1, keepdims=True)
    sel1 = lane == i1
    g2 = jnp.where(sel1, -1.0, gates)
    v2 = jnp.max(g2, axis=-1, keepdims=True)
    i2 = jnp.min(jnp.where(g2 == v2, lane, E), axis=-1, keepdims=True)
    sel2 = lane == i2
    s = v1 + v2
    full = jnp.where(sel1, v1 / s, 0.0) + jnp.where(sel2, v2 / s, 0.0)

    moe = jnp.zeros((TB, D), jnp.float32)
    for e in range(E):
        hid = jax.nn.gelu(_bdot(xn, w1_ref[e]) + b1_ref[e])
        eo = _bdot(hid, w2_ref[e]) + b2_ref[e]
        fe = full[:, e][:, None].astype(jnp.bfloat16).astype(jnp.float32)
        eb = eo.astype(jnp.bfloat16).astype(jnp.float32)
        moe = moe + fe * eb
    xp = h + moe

    t1 = jnp.maximum(_bdot(xp, hw1_ref[...]) + hb1_ref[...], 0.0)
    t2 = _bdot(t1, hw2_ref[...]) + hb2_ref[...]
    ph = jax.nn.sigmoid(t2)

    pm = ph * run
    cum_new = cum + pm
    should = cum_new >= THRESH
    just = jnp.logical_and(should, run > 0.5)
    w = jnp.where(just, 1.0 - cum, pm)
    run_new = jnp.where(should, 0.0, run)

    h_out[...] = xp
    cum_out[...] = cum_new
    run_out[...] = run_new
    acc_out[...] = acc + xp * w
    w_out[...] = w


def _layer_call(first, h, cum, run, acc, pos, lng, lnb, rw, rb, w1, b1,
                w2, b2, hw1, hb1, hw2, hb2):
    blk_nd = pl.BlockSpec((TB, D), lambda i: (i, 0))
    blk_n1 = pl.BlockSpec((TB, 1), lambda i: (i, 0))
    const = lambda shape: pl.BlockSpec(shape, lambda i: tuple(0 for _ in shape))
    out_shapes = (
        jax.ShapeDtypeStruct((NSEQ, D), jnp.float32),
        jax.ShapeDtypeStruct((NSEQ, 1), jnp.float32),
        jax.ShapeDtypeStruct((NSEQ, 1), jnp.float32),
        jax.ShapeDtypeStruct((NSEQ, D), jnp.float32),
        jax.ShapeDtypeStruct((NSEQ, 1), jnp.float32),
    )
    return pl.pallas_call(
        functools.partial(_layer_body, first),
        grid=(NB,),
        in_specs=[
            blk_nd,
            blk_nd,
            blk_n1, blk_n1,
            blk_nd,
            const((1, D)), const((1, D)),
            const((D, E)), const((1, E)),
            const((E, D, DFF)), const((E, 1, DFF)),
            const((E, DFF, D)), const((E, 1, D)),
            const((D, H)), const((1, H)),
            const((H, 1)), const((1, 1)),
        ],
        out_specs=(blk_nd, blk_n1, blk_n1, blk_nd, blk_n1),
        out_shape=out_shapes,
        compiler_params=pltpu.CompilerParams(
            dimension_semantics=("parallel",)),
    )(h, pos, cum, run, acc, lng, lnb, rw, rb, w1, b1, w2, b2, hw1, hb1,
      hw2, hb2)


def _final_body(acc_ref, g_ref, b_ref, w0_ref, w1_ref, w2_ref, w3_ref,
                hf_out, pond_out):
    i = pl.program_id(0)
    acc = acc_ref[...]
    m = jnp.mean(acc, axis=-1, keepdims=True)
    c = acc - m
    v = jnp.mean(c * c, axis=-1, keepdims=True)
    hf = c / jnp.sqrt(v + 1e-5) * g_ref[...] + b_ref[...]
    hf_out[...] = hf.astype(jnp.bfloat16)
    s = jnp.sum(w0_ref[...] + w1_ref[...] + w2_ref[...] + w3_ref[...],
                keepdims=True)

    @pl.when(i == 0)
    def _():
        pond_out[...] = jnp.zeros((1, 1), jnp.float32)

    pond_out[...] += s

    @pl.when(i == NB - 1)
    def _():
        pond_out[...] = pond_out[...] / NSEQ


def _final_call(acc, g, b, w0, w1, w2, w3):
    blk_nd = pl.BlockSpec((TB, D), lambda i: (i, 0))
    blk_n1 = pl.BlockSpec((TB, 1), lambda i: (i, 0))
    const = lambda shape: pl.BlockSpec(shape, lambda i: tuple(0 for _ in shape))
    return pl.pallas_call(
        _final_body,
        grid=(NB,),
        in_specs=[blk_nd, const((1, D)), const((1, D)),
                  blk_n1, blk_n1, blk_n1, blk_n1],
        out_specs=(blk_nd, const((1, 1))),
        out_shape=(jax.ShapeDtypeStruct((NSEQ, D), jnp.bfloat16),
                   jax.ShapeDtypeStruct((1, 1), jnp.float32)),
        compiler_params=pltpu.CompilerParams(
            dimension_semantics=("arbitrary",)),
    )(acc, g, b, w0, w1, w2, w3)


def _head_body(hf_ref, w_ref, b_ref, out_ref):
    out_ref[...] = jnp.dot(
        hf_ref[...], w_ref[...],
        preferred_element_type=jnp.float32) + b_ref[...]


def _head_call(hf, w, b):
    return pl.pallas_call(
        _head_body,
        grid=(NVB,),
        in_specs=[
            pl.BlockSpec((NSEQ, D), lambda j: (0, 0)),
            pl.BlockSpec((D, VB), lambda j: (0, j)),
            pl.BlockSpec((1, VB), lambda j: (0, j)),
        ],
        out_specs=pl.BlockSpec((NSEQ, VB), lambda j: (0, j)),
        out_shape=jax.ShapeDtypeStruct((NSEQ, V), jnp.float32),
        compiler_params=pltpu.CompilerParams(
            dimension_semantics=("parallel",)),
    )(hf, w, b)


def _model(gathered, pos_emb, ln_g, ln_b, rW, rb, W1, b1, W2, b2, hW1, hb1,
           hW2, hb2, lnf_g, lnf_b, head_W, head_b):
    h = gathered
    zero1 = jnp.zeros((NSEQ, 1), jnp.float32)
    cum, run, acc = zero1, zero1, jnp.zeros((NSEQ, D), jnp.float32)
    bf = jnp.bfloat16
    ws = []
    for l in range(L):
        h, cum, run, acc, w = _layer_call(
            l == 0, h, cum, run, acc, pos_emb,
            ln_g[l][None, :], ln_b[l][None, :],
            rW[l].astype(bf), rb[l][None, :],
            W1[l].astype(bf), b1[l][:, None, :],
            W2[l].astype(bf), b2[l][:, None, :],
            hW1[l].astype(bf), hb1[l][None, :],
            hW2[l].astype(bf), hb2[l][None, :])
        ws.append(w)
    hf, pond = _final_call(acc, lnf_g[None, :], lnf_b[None, :], *ws)
    logits = _head_call(hf, head_W.astype(bf), head_b[None, :])
    return logits[None], pond[0, 0]


def kernel(x, tok_emb, pos_emb, ln_g, ln_b, rW, rb, W1, b1, W2, b2, hW1,
           hb1, hW2, hb2, lnf_g, lnf_b, head_W, head_b):
    idx = x.astype(jnp.int32)
    gathered = _sc_gather(tok_emb, idx)
    return _model(gathered, pos_emb, ln_g, ln_b, rW, rb, W1, b1, W2, b2,
                  hW1, hb1, hW2, hb2, lnf_g, lnf_b, head_W, head_b)


# speedup vs baseline: 1.0691x; 1.0691x over previous
import functools

import jax
import jax.numpy as jnp
from jax.experimental import pallas as pl
from jax.experimental.pallas import tpu as pltpu
from jax.experimental.pallas import tpu_sc as plsc

V = 32000
D = 1024
L = 4
E = 8
DFF = 512
NSEQ = 2048
H = D // 2
THRESH = 0.99

TB = 256
NB = NSEQ // TB
VB = 1280
NVB = V // VB
GW = 128
SUB = 8
CW = D // SUB


def _bdot(x, w):
    return jnp.dot(x.astype(jnp.bfloat16), w.astype(jnp.bfloat16),
                   preferred_element_type=jnp.float32)


def _sc_gather(tok_emb, idx):
    mesh = plsc.VectorSubcoreMesh(core_axis_name="core", subcore_axis_name="subcore")
    te = tok_emb.reshape(V * SUB, CW)
    idx8 = (idx.reshape(-1, 1) * SUB
            + jnp.arange(SUB, dtype=jnp.int32)).reshape(1, NSEQ * SUB)

    @functools.partial(
        pl.kernel,
        out_type=jax.ShapeDtypeStruct((NSEQ * SUB, CW), tok_emb.dtype),
        mesh=mesh,
    )
    def kern(x_hbm, i_hbm, o_hbm):
        def body(i_vmem, o_vmem):
            pltpu.sync_copy(x_hbm.at[i_vmem.at[0]], o_vmem)

        pltpu.emit_pipeline(
            body,
            grid=(NSEQ * SUB // GW,),
            in_specs=[pl.BlockSpec((1, GW), lambda i: (0, i))],
            out_specs=[pl.BlockSpec((GW, CW), lambda i: (i, 0))],
            core_axis_name=("core", "subcore"),
            dimension_semantics=(pltpu.PARALLEL,),
        )(i_hbm, o_hbm)

    return kern(te, idx8).reshape(NSEQ, D)


def _layer_body(first, emb_ref, pos_ref, cum_ref, run_ref, acc_ref,
                lng_ref, lnb_ref, rw_ref, rb_ref, w1_ref, b1_ref,
                w2_ref, b2_ref, hw1_ref, hb1_ref, hw2_ref, hb2_ref,
                h_out, cum_out, run_out, acc_out, w_out):
    if first:
        h = emb_ref[...] + pos_ref[...]
        cum = jnp.zeros((TB, 1), jnp.float32)
        run = jnp.ones((TB, 1), jnp.float32)
        acc = jnp.zeros((TB, D), jnp.float32)
    else:
        h = emb_ref[...]
        cum = cum_ref[...]
        run = run_ref[...]
        acc = acc_ref[...]

    m = jnp.mean(h, axis=-1, keepdims=True)
    c = h - m
    v = jnp.mean(c * c, axis=-1, keepdims=True)
    xn = c / jnp.sqrt(v + 1e-5) * lng_ref[...] + lnb_ref[...]

    glog = _bdot(xn, rw_ref[...]) + rb_ref[...]
    gmax = jnp.max(glog, axis=-1, keepdims=True)
    ge = jnp.exp(glog - gmax)
    gates = ge / jnp.sum(ge, axis=-1, keepdims=True)

    lane = jax.lax.broadcasted_iota(jnp.int32, (TB, E), 1)
    v1 = jnp.max(gates, axis=-1, keepdims=True)
    i1 = jnp.min(jnp.where(gates == v1, lane, E), axis=-1, keepdims=True)
    sel1 = lane == i1
    g2 = jnp.where(sel1, -1.0, gates)
    v2 = jnp.max(g2, axis=-1, keepdims=True)
    i2 = jnp.min(jnp.where(g2 == v2, lane, E), axis=-1, keepdims=True)
    sel2 = lane == i2
    s = v1 + v2
    full = jnp.where(sel1, v1 / s, 0.0) + jnp.where(sel2, v2 / s, 0.0)

    moe = jnp.zeros((TB, D), jnp.float32)
    for e in range(E):
        hid = jax.nn.gelu(_bdot(xn, w1_ref[e]) + b1_ref[e])
        eo = _bdot(hid, w2_ref[e]) + b2_ref[e]
        fe = full[:, e][:, None].astype(jnp.bfloat16).astype(jnp.float32)
        eb = eo.astype(jnp.bfloat16).astype(jnp.float32)
        moe = moe + fe * eb
    xp = h + moe

    t1 = jnp.maximum(_bdot(xp, hw1_ref[...]) + hb1_ref[...], 0.0)
    t2 = _bdot(t1, hw2_ref[...]) + hb2_ref[...]
    ph = jax.nn.sigmoid(t2)

    pm = ph * run
    cum_new = cum + pm
    should = cum_new >= THRESH
    just = jnp.logical_and(should, run > 0.5)
    w = jnp.where(just, 1.0 - cum, pm)
    run_new = jnp.where(should, 0.0, run)

    h_out[...] = xp
    cum_out[...] = cum_new
    run_out[...] = run_new
    acc_out[...] = acc + xp * w
    w_out[...] = w


def _layer_call(first, h, cum, run, acc, pos, lng, lnb, rw, rb, w1, b1,
                w2, b2, hw1, hb1, hw2, hb2):
    blk_nd = pl.BlockSpec((TB, D), lambda i: (i, 0))
    blk_n1 = pl.BlockSpec((TB, 1), lambda i: (i, 0))
    const = lambda shape: pl.BlockSpec(shape, lambda i: tuple(0 for _ in shape))
    out_shapes = (
        jax.ShapeDtypeStruct((NSEQ, D), jnp.float32),
        jax.ShapeDtypeStruct((NSEQ, 1), jnp.float32),
        jax.ShapeDtypeStruct((NSEQ, 1), jnp.float32),
        jax.ShapeDtypeStruct((NSEQ, D), jnp.float32),
        jax.ShapeDtypeStruct((NSEQ, 1), jnp.float32),
    )
    return pl.pallas_call(
        functools.partial(_layer_body, first),
        grid=(NB,),
        in_specs=[
            blk_nd,
            blk_nd,
            blk_n1, blk_n1,
            blk_nd,
            const((1, D)), const((1, D)),
            const((D, E)), const((1, E)),
            const((E, D, DFF)), const((E, 1, DFF)),
            const((E, DFF, D)), const((E, 1, D)),
            const((D, H)), const((1, H)),
            const((H, 1)), const((1, 1)),
        ],
        out_specs=(blk_nd, blk_n1, blk_n1, blk_nd, blk_n1),
        out_shape=out_shapes,
        compiler_params=pltpu.CompilerParams(
            dimension_semantics=("parallel",)),
    )(h, pos, cum, run, acc, lng, lnb, rw, rb, w1, b1, w2, b2, hw1, hb1,
      hw2, hb2)


def _final_body(acc_ref, g_ref, b_ref, w0_ref, w1_ref, w2_ref, w3_ref,
                hf_out, pond_out):
    i = pl.program_id(0)
    acc = acc_ref[...]
    m = jnp.mean(acc, axis=-1, keepdims=True)
    c = acc - m
    v = jnp.mean(c * c, axis=-1, keepdims=True)
    hf = c / jnp.sqrt(v + 1e-5) * g_ref[...] + b_ref[...]
    hf_out[...] = hf.astype(jnp.bfloat16)
    s = jnp.sum(w0_ref[...] + w1_ref[...] + w2_ref[...] + w3_ref[...],
                keepdims=True)

    @pl.when(i == 0)
    def _():
        pond_out[...] = jnp.zeros((1, 1), jnp.float32)

    pond_out[...] += s

    @pl.when(i == NB - 1)
    def _():
        pond_out[...] = pond_out[...] / NSEQ


def _final_call(acc, g, b, w0, w1, w2, w3):
    blk_nd = pl.BlockSpec((TB, D), lambda i: (i, 0))
    blk_n1 = pl.BlockSpec((TB, 1), lambda i: (i, 0))
    const = lambda shape: pl.BlockSpec(shape, lambda i: tuple(0 for _ in shape))
    return pl.pallas_call(
        _final_body,
        grid=(NB,),
        in_specs=[blk_nd, const((1, D)), const((1, D)),
                  blk_n1, blk_n1, blk_n1, blk_n1],
        out_specs=(blk_nd, const((1, 1))),
        out_shape=(jax.ShapeDtypeStruct((NSEQ, D), jnp.bfloat16),
                   jax.ShapeDtypeStruct((1, 1), jnp.float32)),
        compiler_params=pltpu.CompilerParams(
            dimension_semantics=("arbitrary",)),
    )(acc, g, b, w0, w1, w2, w3)


def _head_body(hf_ref, w_ref, b_ref, out_ref):
    out_ref[...] = jnp.dot(
        hf_ref[...], w_ref[...].astype(jnp.bfloat16),
        preferred_element_type=jnp.float32) + b_ref[...]


def _head_call(hf, w, b):
    return pl.pallas_call(
        _head_body,
        grid=(NVB,),
        in_specs=[
            pl.BlockSpec((NSEQ, D), lambda j: (0, 0)),
            pl.BlockSpec((D, VB), lambda j: (0, j)),
            pl.BlockSpec((1, VB), lambda j: (0, j)),
        ],
        out_specs=pl.BlockSpec((NSEQ, VB), lambda j: (0, j)),
        out_shape=jax.ShapeDtypeStruct((NSEQ, V), jnp.float32),
        compiler_params=pltpu.CompilerParams(
            dimension_semantics=("parallel",)),
    )(hf, w, b)


def _model(gathered, pos_emb, ln_g, ln_b, rW, rb, W1, b1, W2, b2, hW1, hb1,
           hW2, hb2, lnf_g, lnf_b, head_W, head_b):
    h = gathered
    zero1 = jnp.zeros((NSEQ, 1), jnp.float32)
    cum, run, acc = zero1, zero1, jnp.zeros((NSEQ, D), jnp.float32)
    ws = []
    for l in range(L):
        h, cum, run, acc, w = _layer_call(
            l == 0, h, cum, run, acc, pos_emb,
            ln_g[l][None, :], ln_b[l][None, :],
            rW[l], rb[l][None, :],
            W1[l], b1[l][:, None, :],
            W2[l], b2[l][:, None, :],
            hW1[l], hb1[l][None, :],
            hW2[l], hb2[l][None, :])
        ws.append(w)
    hf, pond = _final_call(acc, lnf_g[None, :], lnf_b[None, :], *ws)
    logits = _head_call(hf, head_W, head_b[None, :])
    return logits[None], pond[0, 0]


def kernel(x, tok_emb, pos_emb, ln_g, ln_b, rW, rb, W1, b1, W2, b2, hW1,
           hb1, hW2, hb2, lnf_g, lnf_b, head_W, head_b):
    idx = x.astype(jnp.int32)
    gathered = _sc_gather(tok_emb, idx)
    return _model(gathered, pos_emb, ln_g, ln_b, rW, rb, W1, b1, W2, b2,
                  hW1, hb1, hW2, hb2, lnf_g, lnf_b, head_W, head_b)


# speedup vs baseline: 1.2830x; 1.2001x over previous
import functools

import jax
import jax.numpy as jnp
from jax.experimental import pallas as pl
from jax.experimental.pallas import tpu as pltpu

V = 32000
D = 1024
L = 4
E = 8
DFF = 512
NSEQ = 2048
H = D // 2
THRESH = 0.99

TB = 256
NB = NSEQ // TB
VB = 1280
NVB = V // VB
GW = 512
SUB = 8
CW = D // SUB


def _bdot(x, w):
    return jnp.dot(x.astype(jnp.bfloat16), w.astype(jnp.bfloat16),
                   preferred_element_type=jnp.float32)


def _layer0_body(idx_ref, tok_ref, pos_ref,
                 lng_ref, lnb_ref, rw_ref, rb_ref, w1_ref, b1_ref,
                 w2_ref, b2_ref, hw1_ref, hb1_ref, hw2_ref, hb2_ref,
                 h_out, cum_out, run_out, acc_out, w_out,
                 scratch, sem):
    blk = pl.program_id(0)

    def issue(i, _):
        r = idx_ref[0, blk * TB + i]
        pltpu.make_async_copy(tok_ref.at[r], scratch.at[i], sem).start()
        return 0

    jax.lax.fori_loop(0, TB, issue, 0)

    def wait(i, _):
        r = idx_ref[0, blk * TB + i]
        pltpu.make_async_copy(tok_ref.at[r], scratch.at[i], sem).wait()
        return 0

    jax.lax.fori_loop(0, TB, wait, 0)
    h = scratch[...] + pos_ref[...]
    cum = jnp.zeros((TB, 1), jnp.float32)
    run = jnp.ones((TB, 1), jnp.float32)
    acc = jnp.zeros((TB, D), jnp.float32)
    _layer_tail(h, cum, run, acc,
                lng_ref, lnb_ref, rw_ref, rb_ref, w1_ref, b1_ref,
                w2_ref, b2_ref, hw1_ref, hb1_ref, hw2_ref, hb2_ref,
                h_out, cum_out, run_out, acc_out, w_out)


def _layer_body(emb_ref, cum_ref, run_ref, acc_ref,
                lng_ref, lnb_ref, rw_ref, rb_ref, w1_ref, b1_ref,
                w2_ref, b2_ref, hw1_ref, hb1_ref, hw2_ref, hb2_ref,
                h_out, cum_out, run_out, acc_out, w_out):
    _layer_tail(emb_ref[...], cum_ref[...], run_ref[...], acc_ref[...],
                lng_ref, lnb_ref, rw_ref, rb_ref, w1_ref, b1_ref,
                w2_ref, b2_ref, hw1_ref, hb1_ref, hw2_ref, hb2_ref,
                h_out, cum_out, run_out, acc_out, w_out)


def _layer_tail(h, cum, run, acc,
                lng_ref, lnb_ref, rw_ref, rb_ref, w1_ref, b1_ref,
                w2_ref, b2_ref, hw1_ref, hb1_ref, hw2_ref, hb2_ref,
                h_out, cum_out, run_out, acc_out, w_out):
    m = jnp.mean(h, axis=-1, keepdims=True)
    c = h - m
    v = jnp.mean(c * c, axis=-1, keepdims=True)
    xn = c / jnp.sqrt(v + 1e-5) * lng_ref[...] + lnb_ref[...]

    glog = _bdot(xn, rw_ref[...]) + rb_ref[...]
    gmax = jnp.max(glog, axis=-1, keepdims=True)
    ge = jnp.exp(glog - gmax)
    gates = ge / jnp.sum(ge, axis=-1, keepdims=True)

    lane = jax.lax.broadcasted_iota(jnp.int32, (TB, E), 1)
    v1 = jnp.max(gates, axis=-1, keepdims=True)
    i1 = jnp.min(jnp.where(gates == v1, lane, E), axis=-1, keepdims=True)
    sel1 = lane == i1
    g2 = jnp.where(sel1, -1.0, gates)
    v2 = jnp.max(g2, axis=-1, keepdims=True)
    i2 = jnp.min(jnp.where(g2 == v2, lane, E), axis=-1, keepdims=True)
    sel2 = lane == i2
    s = v1 + v2
    full = jnp.where(sel1, v1 / s, 0.0) + jnp.where(sel2, v2 / s, 0.0)

    moe = jnp.zeros((TB, D), jnp.float32)
    for e in range(E):
        hid = jax.nn.gelu(_bdot(xn, w1_ref[e]) + b1_ref[e])
        eo = _bdot(hid, w2_ref[e]) + b2_ref[e]
        fe = full[:, e][:, None].astype(jnp.bfloat16).astype(jnp.float32)
        eb = eo.astype(jnp.bfloat16).astype(jnp.float32)
        moe = moe + fe * eb
    xp = h + moe

    t1 = jnp.maximum(_bdot(xp, hw1_ref[...]) + hb1_ref[...], 0.0)
    t2 = _bdot(t1, hw2_ref[...]) + hb2_ref[...]
    ph = jax.nn.sigmoid(t2)

    pm = ph * run
    cum_new = cum + pm
    should = cum_new >= THRESH
    just = jnp.logical_and(should, run > 0.5)
    w = jnp.where(just, 1.0 - cum, pm)
    run_new = jnp.where(should, 0.0, run)

    h_out[...] = xp
    cum_out[...] = cum_new
    run_out[...] = run_new
    acc_out[...] = acc + xp * w
    w_out[...] = w


def _common_specs():
    const = lambda shape: pl.BlockSpec(shape, lambda i: tuple(0 for _ in shape))
    return [
        const((1, D)), const((1, D)),
        const((D, E)), const((1, E)),
        const((E, D, DFF)), const((E, 1, DFF)),
        const((E, DFF, D)), const((E, 1, D)),
        const((D, H)), const((1, H)),
        const((H, 1)), const((1, 1)),
    ]


_OUT_SHAPES = (
    jax.ShapeDtypeStruct((NSEQ, D), jnp.float32),
    jax.ShapeDtypeStruct((NSEQ, 1), jnp.float32),
    jax.ShapeDtypeStruct((NSEQ, 1), jnp.float32),
    jax.ShapeDtypeStruct((NSEQ, D), jnp.float32),
    jax.ShapeDtypeStruct((NSEQ, 1), jnp.float32),
)

_BLK_ND = pl.BlockSpec((TB, D), lambda i: (i, 0))
_BLK_N1 = pl.BlockSpec((TB, 1), lambda i: (i, 0))


def _layer0_call(idx, tok_emb, pos, lng, lnb, rw, rb, w1, b1, w2, b2,
                 hw1, hb1, hw2, hb2):
    blk_nd = pl.BlockSpec((TB, D), lambda i, s: (i, 0))
    blk_n1 = pl.BlockSpec((TB, 1), lambda i, s: (i, 0))
    const = lambda shape: pl.BlockSpec(
        shape, lambda i, s, _shape=shape: tuple(0 for _ in _shape))
    grid_spec = pltpu.PrefetchScalarGridSpec(
        num_scalar_prefetch=1,
        grid=(NB,),
        in_specs=[
            pl.BlockSpec(memory_space=pl.ANY),
            blk_nd,
            const((1, D)), const((1, D)),
            const((D, E)), const((1, E)),
            const((E, D, DFF)), const((E, 1, DFF)),
            const((E, DFF, D)), const((E, 1, D)),
            const((D, H)), const((1, H)),
            const((H, 1)), const((1, 1)),
        ],
        out_specs=list((blk_nd, blk_n1, blk_n1, blk_nd, blk_n1)),
        scratch_shapes=[pltpu.VMEM((TB, D), jnp.float32),
                        pltpu.SemaphoreType.DMA],
    )
    return pl.pallas_call(
        _layer0_body,
        grid_spec=grid_spec,
        out_shape=_OUT_SHAPES,
        compiler_params=pltpu.CompilerParams(
            dimension_semantics=("arbitrary",)),
    )(idx, tok_emb, pos, lng, lnb, rw, rb, w1, b1, w2, b2, hw1, hb1,
      hw2, hb2)


def _layer_call(h, cum, run, acc, lng, lnb, rw, rb, w1, b1, w2, b2,
                hw1, hb1, hw2, hb2):
    return pl.pallas_call(
        _layer_body,
        grid=(NB,),
        in_specs=[_BLK_ND, _BLK_N1, _BLK_N1, _BLK_ND] + _common_specs(),
        out_specs=(_BLK_ND, _BLK_N1, _BLK_N1, _BLK_ND, _BLK_N1),
        out_shape=_OUT_SHAPES,
        compiler_params=pltpu.CompilerParams(
            dimension_semantics=("arbitrary",)),
    )(h, cum, run, acc, lng, lnb, rw, rb, w1, b1, w2, b2, hw1, hb1,
      hw2, hb2)


def _final_body(acc_ref, g_ref, b_ref, w0_ref, w1_ref, w2_ref, w3_ref,
                hf_out, pond_out):
    i = pl.program_id(0)
    acc = acc_ref[...]
    m = jnp.mean(acc, axis=-1, keepdims=True)
    c = acc - m
    v = jnp.mean(c * c, axis=-1, keepdims=True)
    hf = c / jnp.sqrt(v + 1e-5) * g_ref[...] + b_ref[...]
    hf_out[...] = hf.astype(jnp.bfloat16)
    s = jnp.sum(w0_ref[...] + w1_ref[...] + w2_ref[...] + w3_ref[...],
                keepdims=True)

    @pl.when(i == 0)
    def _():
        pond_out[...] = jnp.zeros((1, 1), jnp.float32)

    pond_out[...] += s

    @pl.when(i == NB - 1)
    def _():
        pond_out[...] = pond_out[...] / NSEQ


def _final_call(acc, g, b, w0, w1, w2, w3):
    blk_nd = pl.BlockSpec((TB, D), lambda i: (i, 0))
    blk_n1 = pl.BlockSpec((TB, 1), lambda i: (i, 0))
    const = lambda shape: pl.BlockSpec(shape, lambda i: tuple(0 for _ in shape))
    return pl.pallas_call(
        _final_body,
        grid=(NB,),
        in_specs=[blk_nd, const((1, D)), const((1, D)),
                  blk_n1, blk_n1, blk_n1, blk_n1],
        out_specs=(blk_nd, const((1, 1))),
        out_shape=(jax.ShapeDtypeStruct((NSEQ, D), jnp.bfloat16),
                   jax.ShapeDtypeStruct((1, 1), jnp.float32)),
        compiler_params=pltpu.CompilerParams(
            dimension_semantics=("arbitrary",)),
    )(acc, g, b, w0, w1, w2, w3)


def _head_body(hf_ref, w_ref, b_ref, out_ref):
    out_ref[...] = jnp.dot(
        hf_ref[...], w_ref[...].astype(jnp.bfloat16),
        preferred_element_type=jnp.float32) + b_ref[...]


def _head_call(hf, w, b):
    return pl.pallas_call(
        _head_body,
        grid=(NVB,),
        in_specs=[
            pl.BlockSpec((NSEQ, D), lambda j: (0, 0)),
            pl.BlockSpec((D, VB), lambda j: (0, j)),
            pl.BlockSpec((1, VB), lambda j: (0, j)),
        ],
        out_specs=pl.BlockSpec((NSEQ, VB), lambda j: (0, j)),
        out_shape=jax.ShapeDtypeStruct((NSEQ, V), jnp.float32),
        compiler_params=pltpu.CompilerParams(
            dimension_semantics=("parallel",)),
    )(hf, w, b)


def _model(x_idx, tok_emb, pos_emb, ln_g, ln_b, rW, rb, W1, b1, W2, b2,
           hW1, hb1, hW2, hb2, lnf_g, lnf_b, head_W, head_b):
    ws = []
    for l in range(L):
        wargs = (ln_g[l][None, :], ln_b[l][None, :],
                 rW[l], rb[l][None, :],
                 W1[l], b1[l][:, None, :],
                 W2[l], b2[l][:, None, :],
                 hW1[l], hb1[l][None, :],
                 hW2[l], hb2[l][None, :])
        if l == 0:
            h, cum, run, acc, w = _layer0_call(x_idx, tok_emb, pos_emb,
                                               *wargs)
        else:
            h, cum, run, acc, w = _layer_call(h, cum, run, acc, *wargs)
        ws.append(w)
    hf, pond = _final_call(acc, lnf_g[None, :], lnf_b[None, :], *ws)
    logits = _head_call(hf, head_W, head_b[None, :])
    return logits[None], pond[0, 0]


def kernel(x, tok_emb, pos_emb, ln_g, ln_b, rW, rb, W1, b1, W2, b2, hW1,
           hb1, hW2, hb2, lnf_g, lnf_b, head_W, head_b):
    idx = x.astype(jnp.int32).reshape(1, NSEQ)
    return _model(idx, tok_emb, pos_emb, ln_g, ln_b, rW, rb, W1, b1, W2,
                  b2, hW1, hb1, hW2, hb2, lnf_g, lnf_b, head_W, head_b)


# speedup vs baseline: 1.3107x; 1.0216x over previous
import functools

import jax
import jax.numpy as jnp
from jax.experimental import pallas as pl
from jax.experimental.pallas import tpu as pltpu

V = 32000
D = 1024
L = 4
E = 8
DFF = 512
NSEQ = 2048
H = D // 2
THRESH = 0.99

TB = 256
NB = NSEQ // TB
VB = 1280
NVB = V // VB
GW = 512
SUB = 8
CW = D // SUB


def _bdot(x, w):
    return jnp.dot(x, w, preferred_element_type=jnp.float32)


def _layer0_body(idx_ref, tok_ref, pos_ref,
                 lng_ref, lnb_ref, rw_ref, rb_ref, w1_ref, b1_ref,
                 w2_ref, b2_ref, hw1_ref, hb1_ref, hw2_ref, hb2_ref,
                 h_out, cum_out, run_out, acc_out, w_out,
                 scratch, sem):
    blk = pl.program_id(0)

    def issue(i, _):
        r = idx_ref[0, blk * TB + i]
        pltpu.make_async_copy(tok_ref.at[r], scratch.at[i], sem).start()
        return 0

    jax.lax.fori_loop(0, TB, issue, 0)

    def wait(i, _):
        r = idx_ref[0, blk * TB + i]
        pltpu.make_async_copy(tok_ref.at[r], scratch.at[i], sem).wait()
        return 0

    jax.lax.fori_loop(0, TB, wait, 0)
    h = scratch[...] + pos_ref[...]
    cum = jnp.zeros((TB, 1), jnp.float32)
    run = jnp.ones((TB, 1), jnp.float32)
    acc = jnp.zeros((TB, D), jnp.float32)
    _layer_tail(h, cum, run, acc,
                lng_ref, lnb_ref, rw_ref, rb_ref, w1_ref, b1_ref,
                w2_ref, b2_ref, hw1_ref, hb1_ref, hw2_ref, hb2_ref,
                h_out, cum_out, run_out, acc_out, w_out)


def _layer_body(emb_ref, cum_ref, run_ref, acc_ref,
                lng_ref, lnb_ref, rw_ref, rb_ref, w1_ref, b1_ref,
                w2_ref, b2_ref, hw1_ref, hb1_ref, hw2_ref, hb2_ref,
                h_out, cum_out, run_out, acc_out, w_out):
    _layer_tail(emb_ref[...], cum_ref[...], run_ref[...], acc_ref[...],
                lng_ref, lnb_ref, rw_ref, rb_ref, w1_ref, b1_ref,
                w2_ref, b2_ref, hw1_ref, hb1_ref, hw2_ref, hb2_ref,
                h_out, cum_out, run_out, acc_out, w_out)


def _layer_tail(h, cum, run, acc,
                lng_ref, lnb_ref, rw_ref, rb_ref, w1_ref, b1_ref,
                w2_ref, b2_ref, hw1_ref, hb1_ref, hw2_ref, hb2_ref,
                h_out, cum_out, run_out, acc_out, w_out):
    m = jnp.mean(h, axis=-1, keepdims=True)
    c = h - m
    v = jnp.mean(c * c, axis=-1, keepdims=True)
    xn = c / jnp.sqrt(v + 1e-5) * lng_ref[...] + lnb_ref[...]

    glog = _bdot(xn, rw_ref[...]) + rb_ref[...]
    gmax = jnp.max(glog, axis=-1, keepdims=True)
    ge = jnp.exp(glog - gmax)
    gates = ge / jnp.sum(ge, axis=-1, keepdims=True)

    lane = jax.lax.broadcasted_iota(jnp.int32, (TB, E), 1)
    v1 = jnp.max(gates, axis=-1, keepdims=True)
    i1 = jnp.min(jnp.where(gates == v1, lane, E), axis=-1, keepdims=True)
    sel1 = lane == i1
    g2 = jnp.where(sel1, -1.0, gates)
    v2 = jnp.max(g2, axis=-1, keepdims=True)
    i2 = jnp.min(jnp.where(g2 == v2, lane, E), axis=-1, keepdims=True)
    sel2 = lane == i2
    s = v1 + v2
    full = jnp.where(sel1, v1 / s, 0.0) + jnp.where(sel2, v2 / s, 0.0)

    moe = jnp.zeros((TB, D), jnp.float32)
    for e in range(E):
        hid = jax.nn.gelu(_bdot(xn, w1_ref[e]) + b1_ref[e])
        eo = _bdot(hid, w2_ref[e]) + b2_ref[e]
        fe = full[:, e][:, None].astype(jnp.bfloat16).astype(jnp.float32)
        eb = eo.astype(jnp.bfloat16).astype(jnp.float32)
        moe = moe + fe * eb
    xp = h + moe

    t1 = jnp.maximum(_bdot(xp, hw1_ref[...]) + hb1_ref[...], 0.0)
    t2 = _bdot(t1, hw2_ref[...]) + hb2_ref[...]
    ph = jax.nn.sigmoid(t2)

    pm = ph * run
    cum_new = cum + pm
    should = cum_new >= THRESH
    just = jnp.logical_and(should, run > 0.5)
    w = jnp.where(just, 1.0 - cum, pm)
    run_new = jnp.where(should, 0.0, run)

    h_out[...] = xp
    cum_out[...] = cum_new
    run_out[...] = run_new
    acc_out[...] = acc + xp * w
    w_out[...] = w


def _common_specs():
    const = lambda shape: pl.BlockSpec(shape, lambda i: tuple(0 for _ in shape))
    return [
        const((1, D)), const((1, D)),
        const((D, E)), const((1, E)),
        const((E, D, DFF)), const((E, 1, DFF)),
        const((E, DFF, D)), const((E, 1, D)),
        const((D, H)), const((1, H)),
        const((H, 1)), const((1, 1)),
    ]


_OUT_SHAPES = (
    jax.ShapeDtypeStruct((NSEQ, D), jnp.float32),
    jax.ShapeDtypeStruct((NSEQ, 1), jnp.float32),
    jax.ShapeDtypeStruct((NSEQ, 1), jnp.float32),
    jax.ShapeDtypeStruct((NSEQ, D), jnp.float32),
    jax.ShapeDtypeStruct((NSEQ, 1), jnp.float32),
)

_BLK_ND = pl.BlockSpec((TB, D), lambda i: (i, 0))
_BLK_N1 = pl.BlockSpec((TB, 1), lambda i: (i, 0))


def _layer0_call(idx, tok_emb, pos, lng, lnb, rw, rb, w1, b1, w2, b2,
                 hw1, hb1, hw2, hb2):
    blk_nd = pl.BlockSpec((TB, D), lambda i, s: (i, 0))
    blk_n1 = pl.BlockSpec((TB, 1), lambda i, s: (i, 0))
    const = lambda shape: pl.BlockSpec(
        shape, lambda i, s, _shape=shape: tuple(0 for _ in _shape))
    grid_spec = pltpu.PrefetchScalarGridSpec(
        num_scalar_prefetch=1,
        grid=(NB,),
        in_specs=[
            pl.BlockSpec(memory_space=pl.ANY),
            blk_nd,
            const((1, D)), const((1, D)),
            const((D, E)), const((1, E)),
            const((E, D, DFF)), const((E, 1, DFF)),
            const((E, DFF, D)), const((E, 1, D)),
            const((D, H)), const((1, H)),
            const((H, 1)), const((1, 1)),
        ],
        out_specs=list((blk_nd, blk_n1, blk_n1, blk_nd, blk_n1)),
        scratch_shapes=[pltpu.VMEM((TB, D), jnp.float32),
                        pltpu.SemaphoreType.DMA],
    )
    return pl.pallas_call(
        _layer0_body,
        grid_spec=grid_spec,
        out_shape=_OUT_SHAPES,
        compiler_params=pltpu.CompilerParams(
            dimension_semantics=("arbitrary",)),
    )(idx, tok_emb, pos, lng, lnb, rw, rb, w1, b1, w2, b2, hw1, hb1,
      hw2, hb2)


def _layer_call(h, cum, run, acc, lng, lnb, rw, rb, w1, b1, w2, b2,
                hw1, hb1, hw2, hb2):
    return pl.pallas_call(
        _layer_body,
        grid=(NB,),
        in_specs=[_BLK_ND, _BLK_N1, _BLK_N1, _BLK_ND] + _common_specs(),
        out_specs=(_BLK_ND, _BLK_N1, _BLK_N1, _BLK_ND, _BLK_N1),
        out_shape=_OUT_SHAPES,
        compiler_params=pltpu.CompilerParams(
            dimension_semantics=("arbitrary",)),
    )(h, cum, run, acc, lng, lnb, rw, rb, w1, b1, w2, b2, hw1, hb1,
      hw2, hb2)


def _final_body(acc_ref, g_ref, b_ref, w0_ref, w1_ref, w2_ref, w3_ref,
                hf_out, pond_out):
    i = pl.program_id(0)
    acc = acc_ref[...]
    m = jnp.mean(acc, axis=-1, keepdims=True)
    c = acc - m
    v = jnp.mean(c * c, axis=-1, keepdims=True)
    hf = c / jnp.sqrt(v + 1e-5) * g_ref[...] + b_ref[...]
    hf_out[...] = hf.astype(jnp.bfloat16)
    s = jnp.sum(w0_ref[...] + w1_ref[...] + w2_ref[...] + w3_ref[...],
                keepdims=True)

    @pl.when(i == 0)
    def _():
        pond_out[...] = jnp.zeros((1, 1), jnp.float32)

    pond_out[...] += s

    @pl.when(i == NB - 1)
    def _():
        pond_out[...] = pond_out[...] / NSEQ


def _final_call(acc, g, b, w0, w1, w2, w3):
    blk_nd = pl.BlockSpec((TB, D), lambda i: (i, 0))
    blk_n1 = pl.BlockSpec((TB, 1), lambda i: (i, 0))
    const = lambda shape: pl.BlockSpec(shape, lambda i: tuple(0 for _ in shape))
    return pl.pallas_call(
        _final_body,
        grid=(NB,),
        in_specs=[blk_nd, const((1, D)), const((1, D)),
                  blk_n1, blk_n1, blk_n1, blk_n1],
        out_specs=(blk_nd, const((1, 1))),
        out_shape=(jax.ShapeDtypeStruct((NSEQ, D), jnp.bfloat16),
                   jax.ShapeDtypeStruct((1, 1), jnp.float32)),
        compiler_params=pltpu.CompilerParams(
            dimension_semantics=("arbitrary",)),
    )(acc, g, b, w0, w1, w2, w3)


def _head_body(hf_ref, w_ref, b_ref, out_ref):
    out_ref[...] = jnp.dot(
        hf_ref[...], w_ref[...].astype(jnp.bfloat16),
        preferred_element_type=jnp.float32) + b_ref[...]


def _head_call(hf, w, b):
    return pl.pallas_call(
        _head_body,
        grid=(NVB,),
        in_specs=[
            pl.BlockSpec((NSEQ, D), lambda j: (0, 0)),
            pl.BlockSpec((D, VB), lambda j: (0, j)),
            pl.BlockSpec((1, VB), lambda j: (0, j)),
        ],
        out_specs=pl.BlockSpec((NSEQ, VB), lambda j: (0, j)),
        out_shape=jax.ShapeDtypeStruct((NSEQ, V), jnp.float32),
        compiler_params=pltpu.CompilerParams(
            dimension_semantics=("parallel",)),
    )(hf, w, b)


def _model(x_idx, tok_emb, pos_emb, ln_g, ln_b, rW, rb, W1, b1, W2, b2,
           hW1, hb1, hW2, hb2, lnf_g, lnf_b, head_W, head_b):
    ws = []
    for l in range(L):
        wargs = (ln_g[l][None, :], ln_b[l][None, :],
                 rW[l], rb[l][None, :],
                 W1[l], b1[l][:, None, :],
                 W2[l], b2[l][:, None, :],
                 hW1[l], hb1[l][None, :],
                 hW2[l], hb2[l][None, :])
        if l == 0:
            h, cum, run, acc, w = _layer0_call(x_idx, tok_emb, pos_emb,
                                               *wargs)
        else:
            h, cum, run, acc, w = _layer_call(h, cum, run, acc, *wargs)
        ws.append(w)
    hf, pond = _final_call(acc, lnf_g[None, :], lnf_b[None, :], *ws)
    logits = _head_call(hf, head_W, head_b[None, :])
    return logits[None], pond[0, 0]


def kernel(x, tok_emb, pos_emb, ln_g, ln_b, rW, rb, W1, b1, W2, b2, hW1,
           hb1, hW2, hb2, lnf_g, lnf_b, head_W, head_b):
    idx = x.astype(jnp.int32).reshape(1, NSEQ)
    return _model(idx, tok_emb, pos_emb, ln_g, ln_b, rW, rb, W1, b1, W2,
                  b2, hW1, hb1, hW2, hb2, lnf_g, lnf_b, head_W, head_b)


# speedup vs baseline: 1.5834x; 1.2080x over previous
import functools

import jax
import jax.numpy as jnp
from jax.experimental import pallas as pl
from jax.experimental.pallas import tpu as pltpu

V = 32000
D = 1024
L = 4
E = 8
DFF = 512
NSEQ = 2048
H = D // 2
THRESH = 0.99

TB = 256
NB = NSEQ // TB
VB = 1280
NVB = V // VB
GW = 512
SUB = 8
CW = D // SUB


def _bdot(x, w):
    return jnp.dot(x, w, preferred_element_type=jnp.float32)


def _layer0_body(idx_ref, tok_ref, pos_ref,
                 lng_ref, lnb_ref, rw_ref, rb_ref, w1_ref, b1_ref,
                 w2_ref, b2_ref, hw1_ref, hb1_ref, hw2_ref, hb2_ref,
                 h_out, cum_out, run_out, acc_out, w_out,
                 scratch, sem):
    blk = pl.program_id(0)

    def issue(i, _):
        r = idx_ref[0, blk * TB + i]
        pltpu.make_async_copy(tok_ref.at[r], scratch.at[i], sem).start()
        return 0

    jax.lax.fori_loop(0, TB, issue, 0)

    def wait(i, _):
        r = idx_ref[0, blk * TB + i]
        pltpu.make_async_copy(tok_ref.at[r], scratch.at[i], sem).wait()
        return 0

    jax.lax.fori_loop(0, TB, wait, 0)
    h = scratch[...] + pos_ref[...]
    cum = jnp.zeros((TB, 1), jnp.float32)
    run = jnp.ones((TB, 1), jnp.float32)
    acc = jnp.zeros((TB, D), jnp.float32)
    _layer_tail(h, cum, run, acc,
                lng_ref, lnb_ref, rw_ref, rb_ref, w1_ref, b1_ref,
                w2_ref, b2_ref, hw1_ref, hb1_ref, hw2_ref, hb2_ref,
                h_out, cum_out, run_out, acc_out, w_out)


def _layer_body(emb_ref, cum_ref, run_ref, acc_ref,
                lng_ref, lnb_ref, rw_ref, rb_ref, w1_ref, b1_ref,
                w2_ref, b2_ref, hw1_ref, hb1_ref, hw2_ref, hb2_ref,
                h_out, cum_out, run_out, acc_out, w_out):
    _layer_tail(emb_ref[...], cum_ref[...], run_ref[...], acc_ref[...],
                lng_ref, lnb_ref, rw_ref, rb_ref, w1_ref, b1_ref,
                w2_ref, b2_ref, hw1_ref, hb1_ref, hw2_ref, hb2_ref,
                h_out, cum_out, run_out, acc_out, w_out)


def _layer_tail(h, cum, run, acc,
                lng_ref0, lnb_ref0, rw_ref0, rb_ref0, w1_ref0, b1_ref0,
                w2_ref0, b2_ref0, hw1_ref0, hb1_ref0, hw2_ref0, hb2_ref0,
                h_out, cum_out, run_out, acc_out, w_out):
    lng_ref, lnb_ref, rw_ref, rb_ref = (lng_ref0.at[0], lnb_ref0.at[0],
                                        rw_ref0.at[0], rb_ref0.at[0])
    w1_ref, b1_ref, w2_ref, b2_ref = (w1_ref0.at[0], b1_ref0.at[0],
                                      w2_ref0.at[0], b2_ref0.at[0])
    hw1_ref, hb1_ref, hw2_ref, hb2_ref = (hw1_ref0.at[0], hb1_ref0.at[0],
                                          hw2_ref0.at[0], hb2_ref0.at[0])
    m = jnp.mean(h, axis=-1, keepdims=True)
    c = h - m
    v = jnp.mean(c * c, axis=-1, keepdims=True)
    xn = c / jnp.sqrt(v + 1e-5) * lng_ref[...] + lnb_ref[...]

    glog = _bdot(xn, rw_ref[...]) + rb_ref[...]
    gmax = jnp.max(glog, axis=-1, keepdims=True)
    ge = jnp.exp(glog - gmax)
    gates = ge / jnp.sum(ge, axis=-1, keepdims=True)

    lane = jax.lax.broadcasted_iota(jnp.int32, (TB, E), 1)
    v1 = jnp.max(gates, axis=-1, keepdims=True)
    i1 = jnp.min(jnp.where(gates == v1, lane, E), axis=-1, keepdims=True)
    sel1 = lane == i1
    g2 = jnp.where(sel1, -1.0, gates)
    v2 = jnp.max(g2, axis=-1, keepdims=True)
    i2 = jnp.min(jnp.where(g2 == v2, lane, E), axis=-1, keepdims=True)
    sel2 = lane == i2
    s = v1 + v2
    full = jnp.where(sel1, v1 / s, 0.0) + jnp.where(sel2, v2 / s, 0.0)

    moe = jnp.zeros((TB, D), jnp.float32)
    for e in range(E):
        hid = jax.nn.gelu(_bdot(xn, w1_ref[e]) + b1_ref[e])
        eo = _bdot(hid, w2_ref[e]) + b2_ref[e]
        fe = full[:, e][:, None].astype(jnp.bfloat16).astype(jnp.float32)
        eb = eo.astype(jnp.bfloat16).astype(jnp.float32)
        moe = moe + fe * eb
    xp = h + moe

    t1 = jnp.maximum(_bdot(xp, hw1_ref[...]) + hb1_ref[...], 0.0)
    t2 = _bdot(t1, hw2_ref[...]) + hb2_ref[...]
    ph = jax.nn.sigmoid(t2)

    pm = ph * run
    cum_new = cum + pm
    should = cum_new >= THRESH
    just = jnp.logical_and(should, run > 0.5)
    w = jnp.where(just, 1.0 - cum, pm)
    run_new = jnp.where(should, 0.0, run)

    h_out[...] = xp
    cum_out[...] = cum_new
    run_out[...] = run_new
    acc_out[...] = acc + xp * w
    w_out[...] = w


def _common_specs(l, prefetch):
    if prefetch:
        sel = lambda shape: pl.BlockSpec(
            (1,) + shape, lambda i, s, _n=len(shape): (l,) + (0,) * _n)
    else:
        sel = lambda shape: pl.BlockSpec(
            (1,) + shape, lambda i, _n=len(shape): (l,) + (0,) * _n)
    return [
        sel((1, D)), sel((1, D)),
        sel((D, E)), sel((1, E)),
        sel((E, D, DFF)), sel((E, 1, DFF)),
        sel((E, DFF, D)), sel((E, 1, D)),
        sel((D, H)), sel((1, H)),
        sel((H, 1)), sel((1, 1)),
    ]


_OUT_SHAPES = (
    jax.ShapeDtypeStruct((NSEQ, D), jnp.float32),
    jax.ShapeDtypeStruct((NSEQ, 1), jnp.float32),
    jax.ShapeDtypeStruct((NSEQ, 1), jnp.float32),
    jax.ShapeDtypeStruct((NSEQ, D), jnp.float32),
    jax.ShapeDtypeStruct((NSEQ, 1), jnp.float32),
)

_BLK_ND = pl.BlockSpec((TB, D), lambda i: (i, 0))
_BLK_N1 = pl.BlockSpec((TB, 1), lambda i: (i, 0))


def _layer0_call(idx, tok_emb, pos, lng, lnb, rw, rb, w1, b1, w2, b2,
                 hw1, hb1, hw2, hb2):
    blk_nd = pl.BlockSpec((TB, D), lambda i, s: (i, 0))
    blk_n1 = pl.BlockSpec((TB, 1), lambda i, s: (i, 0))
    grid_spec = pltpu.PrefetchScalarGridSpec(
        num_scalar_prefetch=1,
        grid=(NB,),
        in_specs=[
            pl.BlockSpec(memory_space=pl.ANY),
            blk_nd,
        ] + _common_specs(0, True),
        out_specs=list((blk_nd, blk_n1, blk_n1, blk_nd, blk_n1)),
        scratch_shapes=[pltpu.VMEM((TB, D), jnp.float32),
                        pltpu.SemaphoreType.DMA],
    )
    return pl.pallas_call(
        _layer0_body,
        grid_spec=grid_spec,
        out_shape=_OUT_SHAPES,
        compiler_params=pltpu.CompilerParams(
            dimension_semantics=("arbitrary",)),
    )(idx, tok_emb, pos, lng, lnb, rw, rb, w1, b1, w2, b2, hw1, hb1,
      hw2, hb2)


def _layer_call(l, h, cum, run, acc, lng, lnb, rw, rb, w1, b1, w2, b2,
                hw1, hb1, hw2, hb2):
    return pl.pallas_call(
        _layer_body,
        grid=(NB,),
        in_specs=[_BLK_ND, _BLK_N1, _BLK_N1, _BLK_ND] + _common_specs(l, False),
        out_specs=(_BLK_ND, _BLK_N1, _BLK_N1, _BLK_ND, _BLK_N1),
        out_shape=_OUT_SHAPES,
        compiler_params=pltpu.CompilerParams(
            dimension_semantics=("arbitrary",)),
    )(h, cum, run, acc, lng, lnb, rw, rb, w1, b1, w2, b2, hw1, hb1,
      hw2, hb2)


def _final_body(acc_ref, g_ref, b_ref, w0_ref, w1_ref, w2_ref, w3_ref,
                hf_out, pond_out):
    i = pl.program_id(0)
    acc = acc_ref[...]
    m = jnp.mean(acc, axis=-1, keepdims=True)
    c = acc - m
    v = jnp.mean(c * c, axis=-1, keepdims=True)
    hf = c / jnp.sqrt(v + 1e-5) * g_ref[...] + b_ref[...]
    hf_out[...] = hf.astype(jnp.bfloat16)
    s = jnp.sum(w0_ref[...] + w1_ref[...] + w2_ref[...] + w3_ref[...],
                keepdims=True)

    @pl.when(i == 0)
    def _():
        pond_out[...] = jnp.zeros((1, 1), jnp.float32)

    pond_out[...] += s

    @pl.when(i == NB - 1)
    def _():
        pond_out[...] = pond_out[...] / NSEQ


def _final_call(acc, g, b, w0, w1, w2, w3):
    blk_nd = pl.BlockSpec((TB, D), lambda i: (i, 0))
    blk_n1 = pl.BlockSpec((TB, 1), lambda i: (i, 0))
    const = lambda shape: pl.BlockSpec(shape, lambda i: tuple(0 for _ in shape))
    return pl.pallas_call(
        _final_body,
        grid=(NB,),
        in_specs=[blk_nd, const((1, D)), const((1, D)),
                  blk_n1, blk_n1, blk_n1, blk_n1],
        out_specs=(blk_nd, const((1, 1))),
        out_shape=(jax.ShapeDtypeStruct((NSEQ, D), jnp.bfloat16),
                   jax.ShapeDtypeStruct((1, 1), jnp.float32)),
        compiler_params=pltpu.CompilerParams(
            dimension_semantics=("arbitrary",)),
    )(acc, g, b, w0, w1, w2, w3)


def _head_body(hf_ref, w_ref, b_ref, out_ref):
    out_ref[...] = jnp.dot(
        hf_ref[...], w_ref[...].astype(jnp.bfloat16),
        preferred_element_type=jnp.float32) + b_ref[...]


def _head_call(hf, w, b):
    return pl.pallas_call(
        _head_body,
        grid=(NVB,),
        in_specs=[
            pl.BlockSpec((NSEQ, D), lambda j: (0, 0)),
            pl.BlockSpec((D, VB), lambda j: (0, j)),
            pl.BlockSpec((1, VB), lambda j: (0, j)),
        ],
        out_specs=pl.BlockSpec((NSEQ, VB), lambda j: (0, j)),
        out_shape=jax.ShapeDtypeStruct((NSEQ, V), jnp.float32),
        compiler_params=pltpu.CompilerParams(
            dimension_semantics=("parallel",)),
    )(hf, w, b)


def _model(x_idx, tok_emb, pos_emb, ln_g, ln_b, rW, rb, W1, b1, W2, b2,
           hW1, hb1, hW2, hb2, lnf_g, lnf_b, head_W, head_b):
    ws = []
    wargs = (ln_g[:, None, :], ln_b[:, None, :],
             rW, rb[:, None, :],
             W1, b1[:, :, None, :],
             W2, b2[:, :, None, :],
             hW1, hb1[:, None, :],
             hW2, hb2[:, :, None])
    for l in range(L):
        if l == 0:
            h, cum, run, acc, w = _layer0_call(x_idx, tok_emb, pos_emb,
                                               *wargs)
        else:
            h, cum, run, acc, w = _layer_call(l, h, cum, run, acc, *wargs)
        ws.append(w)
    hf, pond = _final_call(acc, lnf_g[None, :], lnf_b[None, :], *ws)
    logits = _head_call(hf, head_W, head_b[None, :])
    return logits[None], pond[0, 0]


def kernel(x, tok_emb, pos_emb, ln_g, ln_b, rW, rb, W1, b1, W2, b2, hW1,
           hb1, hW2, hb2, lnf_g, lnf_b, head_W, head_b):
    idx = x.astype(jnp.int32).reshape(1, NSEQ)
    return _model(idx, tok_emb, pos_emb, ln_g, ln_b, rW, rb, W1, b1, W2,
                  b2, hW1, hb1, hW2, hb2, lnf_g, lnf_b, head_W, head_b)
